# Initial kernel scaffold; baseline (speedup 1.0000x reference)
import functools

import jax
import jax.numpy as jnp
from jax import lax
from jax.experimental import pallas as pl
from jax.experimental.pallas import tpu as pltpu
from jax.experimental.pallas import tpu_sc as plsc

NC = 2
NS = 16
NW = NC * NS

N_NODES = 10000
N_EDGES = 320000
EPW = N_EDGES // NW
ECH = 80
NCH = EPW // ECH
CAP = 10240
DEGW = 16
BATCH = 1024
TRASH = BATCH
N2_PAD = 1056
D_IN = 128
HC = 256


def _sc_phase_a(x, src, dst, zrows, zdeg, ones_deg):
    mesh = plsc.VectorSubcoreMesh(core_axis_name="c", subcore_axis_name="s")
    out_type = (
        jax.ShapeDtypeStruct((NC, N_NODES, D_IN), jnp.float32),
        jax.ShapeDtypeStruct((NC, N_NODES, DEGW), jnp.float32),
        jax.ShapeDtypeStruct((NW, CAP), jnp.int32),
        jax.ShapeDtypeStruct((NW, CAP), jnp.int32),
        jax.ShapeDtypeStruct((NW, 16), jnp.int32),
    )
    scratch = [
        pltpu.VMEM((ECH,), jnp.int32),
        pltpu.VMEM((ECH,), jnp.int32),
        pltpu.VMEM((ECH, D_IN), jnp.float32),
        pltpu.VMEM((ECH, DEGW), jnp.float32),
        pltpu.VMEM((CAP,), jnp.int32),
        pltpu.VMEM((CAP,), jnp.int32),
        pltpu.VMEM((16,), jnp.int32),
        pltpu.VMEM_SHARED((N_NODES, D_IN), jnp.float32),
        pltpu.VMEM_SHARED((N_NODES, DEGW), jnp.float32),
        pltpu.SemaphoreType.DMA,
    ]

    @functools.partial(pl.kernel, out_type=out_type, mesh=mesh,
                       scratch_types=scratch)
    def k(x_hbm, src_hbm, dst_hbm, zr_hbm, zd_hbm, ones_hbm,
          acc_out, deg_out, srcc_out, dstc_out, cnt_out,
          srci_v, dsti_v, rows_v, ones_v, srcc_v, dstc_v, cntv,
          acc_sh, deg_sh, sem):
        cid = lax.axis_index("c")
        sid = lax.axis_index("s")
        wid = sid * NC + cid
        zpt = N_NODES // NS
        pltpu.sync_copy(zr_hbm, acc_sh.at[pl.ds(sid * zpt, zpt)])
        pltpu.sync_copy(zd_hbm, deg_sh.at[pl.ds(sid * zpt, zpt)])
        pltpu.sync_copy(ones_hbm, ones_v)
        plsc.subcore_barrier()

        ebase = wid * EPW

        def chunk(i, cnt):
            base = ebase + i * ECH
            pltpu.sync_copy(src_hbm.at[pl.ds(base, ECH)], srci_v)
            pltpu.sync_copy(dst_hbm.at[pl.ds(base, ECH)], dsti_v)
            pltpu.async_copy(x_hbm.at[srci_v], rows_v, sem).wait()
            pltpu.sync_copy(rows_v, acc_sh.at[dsti_v], add=True)
            pltpu.sync_copy(ones_v, deg_sh.at[dsti_v], add=True)
            for kk in range(ECH // 16):
                s16 = srci_v[pl.ds(kk * 16, 16)]
                d16 = dsti_v[pl.ds(kk * 16, 16)]
                m = d16 < TRASH
                plsc.store_compressed(srcc_v.at[pl.ds(cnt, 16)], s16, mask=m)
                plsc.store_compressed(dstc_v.at[pl.ds(cnt, 16)], d16, mask=m)
                cnt = cnt + jnp.sum(m.astype(jnp.int32))
            return cnt

        cnt = lax.fori_loop(0, NCH, chunk, jnp.int32(0))
        for kk in range(ECH // 16):
            srcc_v[pl.ds(cnt + kk * 16, 16)] = jnp.zeros((16,), jnp.int32)
            dstc_v[pl.ds(cnt + kk * 16, 16)] = jnp.full((16,), TRASH,
                                                        jnp.int32)
        pltpu.sync_copy(srcc_v, srcc_out.at[wid])
        pltpu.sync_copy(dstc_v, dstc_out.at[wid])
        cntv[...] = jnp.full((16,), cnt, jnp.int32)
        pltpu.sync_copy(cntv, cnt_out.at[wid])
        plsc.subcore_barrier()
        opt = N_NODES // NS
        pltpu.sync_copy(acc_sh.at[pl.ds(sid * opt, opt)],
                        acc_out.at[cid, pl.ds(sid * opt, opt)])
        pltpu.sync_copy(deg_sh.at[pl.ds(sid * opt, opt)],
                        deg_out.at[cid, pl.ds(sid * opt, opt)])

    return k(x, src, dst, zrows, zdeg, ones_deg)


def _sc_phase_c(h1, srcc, dstc, cnts, z2):
    mesh = plsc.VectorSubcoreMesh(core_axis_name="c", subcore_axis_name="s")
    out_type = jax.ShapeDtypeStruct((NC, BATCH, HC), jnp.float32)
    scratch = [
        pltpu.VMEM((ECH,), jnp.int32),
        pltpu.VMEM((ECH,), jnp.int32),
        pltpu.VMEM((ECH, HC), jnp.float32),
        pltpu.VMEM((16,), jnp.int32),
        pltpu.VMEM_SHARED((N2_PAD, HC), jnp.float32),
        pltpu.SemaphoreType.DMA,
    ]

    @functools.partial(pl.kernel, out_type=out_type, mesh=mesh,
                       scratch_types=scratch)
    def k(h1_hbm, srcc_hbm, dstc_hbm, cnt_hbm, z2_hbm, acc2_out,
          srcj_v, dstj_v, rows_v, cntv, acc2_sh, sem):
        cid = lax.axis_index("c")
        sid = lax.axis_index("s")
        wid = sid * NC + cid
        zpt = N2_PAD // NS
        pltpu.sync_copy(z2_hbm, acc2_sh.at[pl.ds(sid * zpt, zpt)])
        pltpu.sync_copy(cnt_hbm.at[wid], cntv)
        plsc.subcore_barrier()
        cnt = jnp.max(cntv[...])
        nch = (cnt + (ECH - 1)) // ECH

        def chunk(j, carry):
            pltpu.sync_copy(srcc_hbm.at[wid, pl.ds(j * ECH, ECH)], srcj_v)
            pltpu.sync_copy(dstc_hbm.at[wid, pl.ds(j * ECH, ECH)], dstj_v)
            pltpu.async_copy(h1_hbm.at[srcj_v], rows_v, sem).wait()
            pltpu.sync_copy(rows_v, acc2_sh.at[dstj_v], add=True)
            return carry

        lax.fori_loop(0, nch, chunk, jnp.int32(0))
        plsc.subcore_barrier()
        opt = BATCH // NS
        pltpu.sync_copy(acc2_sh.at[pl.ds(sid * opt, opt)],
                        acc2_out.at[cid, pl.ds(sid * opt, opt)])

    return k(h1, srcc, dstc, cnts, z2)


def _tc_h1(x, acc, deg, Ws1, Wn1, bs1):
    BLK = 1000

    def body(x_ref, a0_ref, a1_ref, d0_ref, d1_ref, ws_ref, wn_ref, b_ref,
             o_ref):
        d = jnp.maximum(d0_ref[...][:, 0:1] + d1_ref[...][:, 0:1], 1.0)
        agg = (a0_ref[...] + a1_ref[...]) / d
        h = jnp.dot(x_ref[...], ws_ref[...],
                    preferred_element_type=jnp.float32)
        h = h + jnp.dot(agg, wn_ref[...], preferred_element_type=jnp.float32)
        o_ref[...] = jnp.maximum(h + b_ref[...], 0.0)

    return pl.pallas_call(
        body,
        grid=(N_NODES // BLK,),
        in_specs=[
            pl.BlockSpec((BLK, D_IN), lambda i: (i, 0)),
            pl.BlockSpec((BLK, D_IN), lambda i: (i, 0)),
            pl.BlockSpec((BLK, D_IN), lambda i: (i, 0)),
            pl.BlockSpec((BLK, DEGW), lambda i: (i, 0)),
            pl.BlockSpec((BLK, DEGW), lambda i: (i, 0)),
            pl.BlockSpec((D_IN, HC), lambda i: (0, 0)),
            pl.BlockSpec((D_IN, HC), lambda i: (0, 0)),
            pl.BlockSpec((1, HC), lambda i: (0, 0)),
        ],
        out_specs=pl.BlockSpec((BLK, HC), lambda i: (i, 0)),
        out_shape=jax.ShapeDtypeStruct((N_NODES, HC), jnp.float32),
    )(x, acc[0], acc[1], deg[0], deg[1], Ws1, Wn1, bs1.reshape(1, -1))


def _tc_final(E, K, M0, M1, Sc, W1p, b1p, W2p, b2p, W3p, b3p, W4f, b4f,
              h1b, a2a, a2b, d0b, d1b, Ws2, Wn2, bs2, Wm1, bm1, Wm2p, bm2p,
              Wm3f):
    BLK = 128
    L = E.shape[1]
    LD = K.shape[1]

    def body(e_ref, k_ref, m0_ref, m1_ref, sc_ref,
             w1_ref, b1_ref, w2_ref, b2_ref, w3_ref, b3_ref, w4_ref, b4_ref,
             h1_ref, a2a_ref, a2b_ref, d0_ref, d1_ref,
             ws2_ref, wn2_ref, bs2_ref, wm1_ref, bm1_ref, wm2_ref, bm2_ref,
             wm3_ref, o_ref):
        dot = lambda a, b: jnp.dot(a, b, preferred_element_type=jnp.float32)
        e = e_ref[...]
        q = dot(e, k_ref[...]) * dot(e, m0_ref[...]) * dot(e, m1_ref[...])
        pooled = dot(q, sc_ref[...])
        h = jnp.maximum(dot(pooled, w1_ref[...]) + b1_ref[...], 0.0)
        h = jnp.maximum(dot(h, w2_ref[...]) + b2_ref[...], 0.0)
        h = jnp.maximum(dot(h, w3_ref[...]) + b3_ref[...], 0.0)
        hy = dot(h, w4_ref[...]) + b4_ref[...]
        d = jnp.maximum(d0_ref[...][:, 0:1] + d1_ref[...][:, 0:1], 1.0)
        agg2 = (a2a_ref[...] + a2b_ref[...]) / d
        g = jnp.maximum(dot(h1_ref[...], ws2_ref[...]) +
                        dot(agg2, wn2_ref[...]) + bs2_ref[...], 0.0)
        m = jnp.maximum(dot(g, wm1_ref[...]) + bm1_ref[...], 0.0)
        m = jnp.maximum(dot(m, wm2_ref[...]) + bm2_ref[...], 0.0)
        o_ref[...] = hy + dot(m, wm3_ref[...])

    full = lambda r, c: pl.BlockSpec((r, c), lambda i: (0, 0))
    return pl.pallas_call(
        body,
        grid=(BATCH // BLK,),
        in_specs=[
            pl.BlockSpec((BLK, L), lambda i: (i, 0)),
            full(L, LD), full(L, LD), full(L, LD), full(LD, 128),
            full(128, 1024), full(1, 1024),
            full(1024, 512), full(1, 512),
            full(512, 256), full(1, 256),
            full(256, 128), full(1, 128),
            pl.BlockSpec((BLK, HC), lambda i: (i, 0)),
            pl.BlockSpec((BLK, HC), lambda i: (i, 0)),
            pl.BlockSpec((BLK, HC), lambda i: (i, 0)),
            pl.BlockSpec((BLK, DEGW), lambda i: (i, 0)),
            pl.BlockSpec((BLK, DEGW), lambda i: (i, 0)),
            full(HC, HC), full(HC, HC), full(1, HC),
            full(HC, 128), full(1, 128),
            full(128, 128), full(1, 128),
            full(128, 128),
        ],
        out_specs=pl.BlockSpec((BLK, 128), lambda i: (i, 0)),
        out_shape=jax.ShapeDtypeStruct((BATCH, 128), jnp.float32),
    )(E, K, M0, M1, Sc, W1p, b1p, W2p, b2p, W3p, b3p, W4f, b4f,
      h1b, a2a, a2b, d0b, d1b, Ws2, Wn2, bs2, Wm1, bm1, Wm2p, bm2p, Wm3f)


def kernel(x, edge_index, inputs_embeds, batch_size, Wm_in, filt, Wg, Wv,
           W1, b1, W2, b2, W3, b3, W4, b4,
           Ws1, Wn1, bs1, Ws2, Wn2, bs2,
           Wm1, bm1, Wm2, bm2, Wm3, bm3, Wfc, bfc):
    f32 = jnp.float32
    src = edge_index[0]
    dst = edge_index[1]

    zrows = jnp.zeros((N_NODES // NS, D_IN), f32)
    zdeg = jnp.zeros((N_NODES // NS, DEGW), f32)
    ones_deg = jnp.ones((ECH, DEGW), f32)
    acc, deg, srcc, dstc, cnts = _sc_phase_a(x, src, dst, zrows, zdeg,
                                             ones_deg)

    h1 = _tc_h1(x, acc, deg, Ws1, Wn1, bs1)

    z2 = jnp.zeros((N2_PAD // NS, HC), f32)
    acc2 = _sc_phase_c(h1, srcc, dstc, cnts, z2)

    L = inputs_embeds.shape[1]
    DM = Wv.shape[0]
    r = jnp.arange(L)
    dmat = r[None, :] - r[:, None]
    f0 = filt[0]
    f1 = filt[1]
    M0 = jnp.where((dmat >= 0)[:, :, None], f0[dmat % L],
                   0.0).reshape(L, L * DM)
    M1 = jnp.where((dmat.T >= 0)[:, :, None], f1[dmat.T % L],
                   0.0).reshape(L, L * DM)
    K = jnp.repeat(jnp.eye(L, dtype=f32), DM, axis=1)
    v = Wm_in @ Wv
    g0 = Wm_in @ Wg[0]
    g1 = Wm_in @ Wg[1]
    coef = g0 * g1 * v / L
    Sc = jnp.pad(jnp.tile(jnp.diag(coef), (L, 1)), ((0, 0), (0, 128 - DM)))
    W1p = jnp.pad(W1, ((0, 128 - DM), (0, 24)))
    b1p = jnp.pad(b1, (0, 24)).reshape(1, -1)
    W2p = jnp.pad(W2, ((0, 24), (0, 12)))
    b2p = jnp.pad(b2, (0, 12)).reshape(1, -1)
    W3p = jnp.pad(W3, ((0, 12), (0, 56)))
    b3p = jnp.pad(b3, (0, 56)).reshape(1, -1)
    wfc0 = Wfc[0, 0]
    wfc1 = Wfc[1, 0]
    W4f = jnp.pad(W4 * wfc0, ((0, 56), (0, 127)))
    b4f = jnp.zeros((1, 128), f32).at[0, 0].set(
        wfc0 * b4[0] + wfc1 * bm3[0] + bfc[0])
    Wm2p = jnp.pad(Wm2, ((0, 0), (0, 64)))
    bm2p = jnp.pad(bm2, (0, 64)).reshape(1, -1)
    Wm3f = jnp.pad(Wm3 * wfc1, ((0, 64), (0, 127)))

    out2d = _tc_final(inputs_embeds, K, M0, M1, Sc, W1p, b1p, W2p, b2p,
                      W3p, b3p, W4f, b4f,
                      h1[:BATCH], acc2[0], acc2[1],
                      deg[0, :BATCH], deg[1, :BATCH],
                      Ws2, Wn2, bs2.reshape(1, -1), Wm1, bm1.reshape(1, -1),
                      Wm2p, bm2p, Wm3f)
    return out2d[:, 0]


# speedup vs baseline: 7.2136x; 7.2136x over previous
import functools

import jax
import jax.numpy as jnp
from jax import lax
from jax.experimental import pallas as pl
from jax.experimental.pallas import tpu as pltpu
from jax.experimental.pallas import tpu_sc as plsc

NC = 2
NS = 16
NW = NC * NS

N_NODES = 10000
N_PAD = 10112
N_EDGES = 320000
EPW = N_EDGES // NW
ECH = 80
NCH = EPW // ECH
CAP = 10240
DEGW = 8
DEG_BASE = N_PAD
IDENT_BASE = N_PAD + 80
N_EXT = 208
N_ALL = N_PAD + N_EXT
A_BASES = tuple(range(0, 560, 80)) + (552,)
EXT_BASES = (0, 80, 128)
BATCH = 1024
TRASH = BATCH
N2_PAD = 1280
D_IN = 128
HC = 256


def _sc_phase_a(x, src, dst, zrows, ident):
    mesh = plsc.VectorSubcoreMesh(core_axis_name="c", subcore_axis_name="s")
    out_type = (
        jax.ShapeDtypeStruct((NC, N_PAD, D_IN), jnp.float32),
        jax.ShapeDtypeStruct((NC, 80, D_IN), jnp.float32),
        jax.ShapeDtypeStruct((NW * CAP,), jnp.int32),
        jax.ShapeDtypeStruct((NW * 16,), jnp.int32),
    )
    scratch = [
        pltpu.VMEM((ECH,), jnp.int32),
        pltpu.VMEM((ECH,), jnp.int32),
        pltpu.VMEM((ECH,), jnp.int32),
        pltpu.VMEM((ECH,), jnp.int32),
        pltpu.VMEM((ECH, D_IN), jnp.float32),
        pltpu.VMEM((ECH, D_IN), jnp.float32),
        pltpu.VMEM((128, D_IN), jnp.float32),
        pltpu.VMEM((CAP,), jnp.int32),
        pltpu.VMEM((16,), jnp.int32),
        pltpu.VMEM((ECH,), jnp.int32),
        pltpu.VMEM((64,), jnp.int32),
        pltpu.VMEM_SHARED((N_ALL, D_IN), jnp.float32),
        pltpu.SemaphoreType.DMA,
        pltpu.SemaphoreType.DMA,
    ]

    @functools.partial(pl.kernel, out_type=out_type, mesh=mesh,
                       scratch_types=scratch,
                       compiler_params=pltpu.CompilerParams(
                           needs_layout_passes=False))
    def k(x_hbm, src_hbm, dst_hbm, zr_hbm, id_hbm,
          acc_out, deg_out, cc_out, cnt_out,
          srci_v, dsti_v, oidx_v, didx_v, rows_v, rows2_v, identv,
          cc_v, cntv, ramp_v, ramp64_v,
          acc_sh, sem, sem2):
        cid = lax.axis_index("c")
        sid = lax.axis_index("s")
        wid = sid * NC + cid
        lane = lax.iota(jnp.int32, 16)
        zpt = N_PAD // NS
        pltpu.sync_copy(zr_hbm, rows_v)

        def fill_ramp(base):
            for kk in range(ECH // 16):
                ramp_v[pl.ds(kk * 16, 16)] = base + kk * 16 + lane

        for q in A_BASES:
            fill_ramp(sid * zpt + q)
            pltpu.sync_copy(rows_v, acc_sh.at[ramp_v])

        @pl.when(sid == 0)
        def _init_ext():
            for q in EXT_BASES:
                fill_ramp(N_PAD + q)
                pltpu.sync_copy(rows_v, acc_sh.at[ramp_v])
            pltpu.sync_copy(id_hbm, identv)
            for half in range(2):
                for kk in range(4):
                    ramp64_v[pl.ds(kk * 16, 16)] = (IDENT_BASE + half * 64 +
                                                    kk * 16 + lane)
                pltpu.sync_copy(identv.at[pl.ds(half * 64, 64)],
                                acc_sh.at[ramp64_v])

        plsc.subcore_barrier()

        ebase = wid * EPW

        def chunk(i, cnt):
            base = ebase + i * ECH
            pltpu.sync_copy(src_hbm.at[pl.ds(base, ECH)], srci_v)
            pltpu.sync_copy(dst_hbm.at[pl.ds(base, ECH)], dsti_v)
            gat = pltpu.async_copy(x_hbm.at[srci_v], rows_v, sem)
            for kk in range(ECH // 16):
                d16 = dsti_v[pl.ds(kk * 16, 16)]
                oidx_v[pl.ds(kk * 16, 16)] = IDENT_BASE + (d16 & 127)
                didx_v[pl.ds(kk * 16, 16)] = (
                    DEG_BASE + lax.shift_right_logical(d16, 7))
            gat2 = pltpu.async_copy(acc_sh.at[oidx_v], rows2_v, sem2)
            gat.wait()
            pltpu.sync_copy(rows_v, acc_sh.at[dsti_v], add=True)
            gat2.wait()
            pltpu.sync_copy(rows2_v, acc_sh.at[didx_v], add=True)
            for kk in range(ECH // 16):
                s16 = srci_v[pl.ds(kk * 16, 16)]
                d16 = dsti_v[pl.ds(kk * 16, 16)]
                m = d16 < TRASH
                pref = plsc.cumsum(m.astype(jnp.int32))
                pos = jnp.where(m, cnt + pref - 1, CAP - 16 + lane)
                pk = lax.shift_left(d16, 18) | s16
                plsc.store_scatter(cc_v, [pos], pk)
                cnt = cnt + jnp.max(pref)
            return cnt

        cnt = lax.fori_loop(0, NCH, chunk, jnp.int32(0))
        for kk in range(ECH // 16):
            plsc.store_scatter(cc_v, [cnt + kk * 16 + lane],
                               jnp.full((16,), TRASH << 18, jnp.int32))
        pltpu.sync_copy(cc_v, cc_out.at[pl.ds(wid * CAP, CAP)])
        cntv[...] = jnp.full((16,), cnt, jnp.int32)
        pltpu.sync_copy(cntv, cnt_out.at[pl.ds(wid * 16, 16)])
        plsc.subcore_barrier()
        opt = N_PAD // NS
        for q in A_BASES:
            fill_ramp(sid * opt + q)
            pltpu.async_copy(acc_sh.at[ramp_v], rows_v, sem).wait()
            pltpu.sync_copy(rows_v,
                            acc_out.at[cid, pl.ds(sid * opt + q, ECH)])

        @pl.when(sid == 0)
        def _read_deg():
            fill_ramp(DEG_BASE)
            pltpu.async_copy(acc_sh.at[ramp_v], rows_v, sem).wait()
            pltpu.sync_copy(rows_v, deg_out.at[cid])

    return k(x, src, dst, zrows, ident)


def _sc_phase_c(h1a, h1b, cc, cnts, z2):
    mesh = plsc.VectorSubcoreMesh(core_axis_name="c", subcore_axis_name="s")
    out_type = (
        jax.ShapeDtypeStruct((NC, BATCH, HC // 2), jnp.float32),
        jax.ShapeDtypeStruct((NC, BATCH, HC // 2), jnp.float32),
    )
    scratch = [
        pltpu.VMEM((ECH,), jnp.int32),
        pltpu.VMEM((ECH,), jnp.int32),
        pltpu.VMEM((ECH,), jnp.int32),
        pltpu.VMEM((ECH, HC // 2), jnp.float32),
        pltpu.VMEM((ECH, HC // 2), jnp.float32),
        pltpu.VMEM((16,), jnp.int32),
        pltpu.VMEM_SHARED((N2_PAD, HC // 2), jnp.float32),
        pltpu.VMEM_SHARED((N2_PAD, HC // 2), jnp.float32),
        pltpu.SemaphoreType.DMA,
        pltpu.SemaphoreType.DMA,
    ]

    @functools.partial(pl.kernel, out_type=out_type, mesh=mesh,
                       scratch_types=scratch,
                       compiler_params=pltpu.CompilerParams(
                           needs_layout_passes=False))
    def k(h1a_hbm, h1b_hbm, cc_hbm, cnt_hbm, z2_hbm,
          acc2a_out, acc2b_out,
          ccj_v, srcj_v, dstj_v, rowsa_v, rowsb_v, cntv, acc2a_sh, acc2b_sh,
          sema, semb):
        cid = lax.axis_index("c")
        sid = lax.axis_index("s")
        wid = sid * NC + cid
        zpt = N2_PAD // NS
        pltpu.sync_copy(z2_hbm, acc2a_sh.at[pl.ds(sid * zpt, zpt)])
        pltpu.sync_copy(z2_hbm, acc2b_sh.at[pl.ds(sid * zpt, zpt)])
        pltpu.sync_copy(cnt_hbm.at[pl.ds(wid * 16, 16)], cntv)
        plsc.subcore_barrier()
        cnt = jnp.max(cntv[...])
        nch = (cnt + (ECH - 1)) // ECH

        def chunk(j, carry):
            pltpu.sync_copy(cc_hbm.at[pl.ds(wid * CAP + j * ECH, ECH)],
                            ccj_v)
            for kk in range(ECH // 16):
                pk = ccj_v[pl.ds(kk * 16, 16)]
                srcj_v[pl.ds(kk * 16, 16)] = pk & ((1 << 18) - 1)
                dstj_v[pl.ds(kk * 16, 16)] = lax.shift_right_logical(pk, 18)
            ca = pltpu.async_copy(h1a_hbm.at[srcj_v], rowsa_v, sema)
            cb = pltpu.async_copy(h1b_hbm.at[srcj_v], rowsb_v, semb)
            ca.wait()
            pltpu.sync_copy(rowsa_v, acc2a_sh.at[dstj_v], add=True)
            cb.wait()
            pltpu.sync_copy(rowsb_v, acc2b_sh.at[dstj_v], add=True)
            return carry

        lax.fori_loop(0, nch, chunk, jnp.int32(0))
        plsc.subcore_barrier()
        opt = BATCH // NS
        pltpu.sync_copy(acc2a_sh.at[pl.ds(sid * opt, opt)],
                        acc2a_out.at[cid, pl.ds(sid * opt, opt)])
        pltpu.sync_copy(acc2b_sh.at[pl.ds(sid * opt, opt)],
                        acc2b_out.at[cid, pl.ds(sid * opt, opt)])

    return k(h1a, h1b, cc, cnts, z2)


def _tc_h1(x, acc, deg, Ws1, Wn1, bs1):
    BLK = 1000
    H2 = HC // 2

    def body(x_ref, a0_ref, a1_ref, d_ref, ws_ref, wn_ref, b_ref,
             oa_ref, ob_ref):
        d = jnp.maximum(d_ref[...][:, 0:1], 1.0)
        agg = (a0_ref[...] + a1_ref[...]) / d
        h = jnp.dot(x_ref[...], ws_ref[...],
                    preferred_element_type=jnp.float32)
        h = h + jnp.dot(agg, wn_ref[...], preferred_element_type=jnp.float32)
        h = jnp.maximum(h + b_ref[...], 0.0)
        oa_ref[...] = h[:, :H2]
        ob_ref[...] = h[:, H2:]

    return pl.pallas_call(
        body,
        grid=(N_NODES // BLK,),
        in_specs=[
            pl.BlockSpec((BLK, D_IN), lambda i: (i, 0)),
            pl.BlockSpec((BLK, D_IN), lambda i: (i, 0)),
            pl.BlockSpec((BLK, D_IN), lambda i: (i, 0)),
            pl.BlockSpec((BLK, DEGW), lambda i: (i, 0)),
            pl.BlockSpec((D_IN, HC), lambda i: (0, 0)),
            pl.BlockSpec((D_IN, HC), lambda i: (0, 0)),
            pl.BlockSpec((1, HC), lambda i: (0, 0)),
        ],
        out_specs=[
            pl.BlockSpec((BLK, H2), lambda i: (i, 0)),
            pl.BlockSpec((BLK, H2), lambda i: (i, 0)),
        ],
        out_shape=[
            jax.ShapeDtypeStruct((N_NODES, H2), jnp.float32),
            jax.ShapeDtypeStruct((N_NODES, H2), jnp.float32),
        ],
    )(x, acc[0], acc[1], deg, Ws1, Wn1, bs1.reshape(1, -1))


def _tc_final(E, K, M0, M1, Sc, W1p, b1p, W2p, b2p, W3p, b3p, W4f, b4f,
              h1a, h1b, a2a0, a2a1, a2b0, a2b1, degb,
              Ws2, Wn2, bs2, Wm1, bm1, Wm2p, bm2p, Wm3f):
    BLK = 128
    L = E.shape[1]
    LD = K.shape[1]
    H2 = HC // 2

    def body(e_ref, k_ref, m0_ref, m1_ref, sc_ref,
             w1_ref, b1_ref, w2_ref, b2_ref, w3_ref, b3_ref, w4_ref, b4_ref,
             h1a_ref, h1b_ref, a2a0_ref, a2a1_ref, a2b0_ref, a2b1_ref,
             d_ref,
             ws2_ref, wn2_ref, bs2_ref, wm1_ref, bm1_ref, wm2_ref, bm2_ref,
             wm3_ref, o_ref):
        dot = lambda a, b: jnp.dot(a, b, preferred_element_type=jnp.float32)
        e = e_ref[...]
        q = dot(e, k_ref[...]) * dot(e, m0_ref[...]) * dot(e, m1_ref[...])
        pooled = dot(q, sc_ref[...])
        h = jnp.maximum(dot(pooled, w1_ref[...]) + b1_ref[...], 0.0)
        h = jnp.maximum(dot(h, w2_ref[...]) + b2_ref[...], 0.0)
        h = jnp.maximum(dot(h, w3_ref[...]) + b3_ref[...], 0.0)
        hy = dot(h, w4_ref[...]) + b4_ref[...]
        d = jnp.maximum(d_ref[...][:, 0:1], 1.0)
        agg2a = (a2a0_ref[...] + a2a1_ref[...]) / d
        agg2b = (a2b0_ref[...] + a2b1_ref[...]) / d
        ws2 = ws2_ref[...]
        wn2 = wn2_ref[...]
        g = (dot(h1a_ref[...], ws2[:H2]) + dot(h1b_ref[...], ws2[H2:]) +
             dot(agg2a, wn2[:H2]) + dot(agg2b, wn2[H2:]) + bs2_ref[...])
        g = jnp.maximum(g, 0.0)
        m = jnp.maximum(dot(g, wm1_ref[...]) + bm1_ref[...], 0.0)
        m = jnp.maximum(dot(m, wm2_ref[...]) + bm2_ref[...], 0.0)
        o_ref[...] = hy + dot(m, wm3_ref[...])

    full = lambda r, c: pl.BlockSpec((r, c), lambda i: (0, 0))
    row = lambda c: pl.BlockSpec((BLK, c), lambda i: (i, 0))
    return pl.pallas_call(
        body,
        grid=(BATCH // BLK,),
        in_specs=[
            row(L),
            full(L, LD), full(L, LD), full(L, LD), full(LD, 128),
            full(128, 1024), full(1, 1024),
            full(1024, 512), full(1, 512),
            full(512, 256), full(1, 256),
            full(256, 128), full(1, 128),
            row(H2), row(H2),
            row(H2), row(H2), row(H2), row(H2),
            row(DEGW),
            full(HC, HC), full(HC, HC), full(1, HC),
            full(HC, 128), full(1, 128),
            full(128, 128), full(1, 128),
            full(128, 128),
        ],
        out_specs=pl.BlockSpec((BLK, 128), lambda i: (i, 0)),
        out_shape=jax.ShapeDtypeStruct((BATCH, 128), jnp.float32),
    )(E, K, M0, M1, Sc, W1p, b1p, W2p, b2p, W3p, b3p, W4f, b4f,
      h1a, h1b, a2a0, a2a1, a2b0, a2b1, degb,
      Ws2, Wn2, bs2, Wm1, bm1, Wm2p, bm2p, Wm3f)


def kernel(x, edge_index, inputs_embeds, batch_size, Wm_in, filt, Wg, Wv,
           W1, b1, W2, b2, W3, b3, W4, b4,
           Ws1, Wn1, bs1, Ws2, Wn2, bs2,
           Wm1, bm1, Wm2, bm2, Wm3, bm3, Wfc, bfc):
    f32 = jnp.float32
    src = edge_index[0]
    dst = edge_index[1]

    zrows = jnp.zeros((ECH, D_IN), f32)
    ident = jnp.eye(128, dtype=f32)
    acc, degblk, cc, cnts = _sc_phase_a(x, src, dst, zrows, ident)
    deg_flat = (degblk[0] + degblk[1]).reshape(-1)[:N_NODES]
    degp = jnp.pad(deg_flat[:, None], ((0, 0), (0, DEGW - 1)))

    h1a, h1b = _tc_h1(x, acc[:, :N_NODES], degp, Ws1, Wn1, bs1)

    z2 = jnp.zeros((N2_PAD // NS, HC // 2), f32)
    acc2a, acc2b = _sc_phase_c(h1a, h1b, cc, cnts, z2)

    L = inputs_embeds.shape[1]
    DM = Wv.shape[0]
    r = jnp.arange(L)
    dmat = r[None, :] - r[:, None]
    f0 = filt[0]
    f1 = filt[1]
    M0 = jnp.where((dmat >= 0)[:, :, None], f0[dmat % L],
                   0.0).reshape(L, L * DM)
    M1 = jnp.where((dmat.T >= 0)[:, :, None], f1[dmat.T % L],
                   0.0).reshape(L, L * DM)
    K = jnp.repeat(jnp.eye(L, dtype=f32), DM, axis=1)
    v = Wm_in @ Wv
    g0 = Wm_in @ Wg[0]
    g1 = Wm_in @ Wg[1]
    coef = g0 * g1 * v / L
    Sc = jnp.pad(jnp.tile(jnp.diag(coef), (L, 1)), ((0, 0), (0, 128 - DM)))
    W1p = jnp.pad(W1, ((0, 128 - DM), (0, 24)))
    b1p = jnp.pad(b1, (0, 24)).reshape(1, -1)
    W2p = jnp.pad(W2, ((0, 24), (0, 12)))
    b2p = jnp.pad(b2, (0, 12)).reshape(1, -1)
    W3p = jnp.pad(W3, ((0, 12), (0, 56)))
    b3p = jnp.pad(b3, (0, 56)).reshape(1, -1)
    wfc0 = Wfc[0, 0]
    wfc1 = Wfc[1, 0]
    W4f = jnp.pad(W4 * wfc0, ((0, 56), (0, 127)))
    b4f = jnp.zeros((1, 128), f32).at[0, 0].set(
        wfc0 * b4[0] + wfc1 * bm3[0] + bfc[0])
    Wm2p = jnp.pad(Wm2, ((0, 0), (0, 64)))
    bm2p = jnp.pad(bm2, (0, 64)).reshape(1, -1)
    Wm3f = jnp.pad(Wm3 * wfc1, ((0, 64), (0, 127)))

    out2d = _tc_final(inputs_embeds, K, M0, M1, Sc, W1p, b1p, W2p, b2p,
                      W3p, b3p, W4f, b4f,
                      h1a[:BATCH], h1b[:BATCH],
                      acc2a[0], acc2a[1], acc2b[0], acc2b[1],
                      degp[:BATCH],
                      Ws2, Wn2, bs2.reshape(1, -1), Wm1, bm1.reshape(1, -1),
                      Wm2p, bm2p, Wm3f)
    return out2d[:, 0]


# speedup vs baseline: 7.3617x; 1.0205x over previous
import functools

import jax
import jax.numpy as jnp
from jax import lax
from jax.experimental import pallas as pl
from jax.experimental.pallas import tpu as pltpu
from jax.experimental.pallas import tpu_sc as plsc

NC = 2
NS = 16
NW = NC * NS

N_NODES = 10000
N_PAD = 10112
N_EDGES = 320000
EPW = N_EDGES // NW
ECH = 80
NCH = EPW // ECH
CAP = 10240
DEGW = 8
DEG_BASE = N_PAD
IDENT_BASE = N_PAD + 80
N_EXT = 208
N_ALL = N_PAD + N_EXT
A_BASES = tuple(range(0, 560, 80)) + (552,)
EXT_BASES = (0, 80, 128)
BATCH = 1024
TRASH = BATCH
N2_PAD = 1280
D_IN = 128
HC = 256


def _sc_phase_a(x, src, dst, zrows, ident):
    mesh = plsc.VectorSubcoreMesh(core_axis_name="c", subcore_axis_name="s")
    out_type = (
        jax.ShapeDtypeStruct((NC, N_PAD, D_IN), jnp.float32),
        jax.ShapeDtypeStruct((NC, 80, D_IN), jnp.float32),
        jax.ShapeDtypeStruct((NW * CAP,), jnp.int32),
        jax.ShapeDtypeStruct((NW * 16,), jnp.int32),
    )
    scratch = [
        pltpu.VMEM((ECH,), jnp.int32),
        pltpu.VMEM((ECH,), jnp.int32),
        pltpu.VMEM((ECH,), jnp.int32),
        pltpu.VMEM((ECH,), jnp.int32),
        pltpu.VMEM((ECH, D_IN), jnp.float32),
        pltpu.VMEM((ECH, D_IN), jnp.float32),
        pltpu.VMEM((128, D_IN), jnp.float32),
        pltpu.VMEM((CAP,), jnp.int32),
        pltpu.VMEM((16,), jnp.int32),
        pltpu.VMEM((ECH,), jnp.int32),
        pltpu.VMEM((64,), jnp.int32),
        pltpu.VMEM_SHARED((N_ALL, D_IN), jnp.float32),
        pltpu.SemaphoreType.DMA,
        pltpu.SemaphoreType.DMA,
        pltpu.SemaphoreType.DMA,
        pltpu.SemaphoreType.DMA,
    ]

    @functools.partial(pl.kernel, out_type=out_type, mesh=mesh,
                       scratch_types=scratch,
                       compiler_params=pltpu.CompilerParams(
                           needs_layout_passes=False))
    def k(x_hbm, src_hbm, dst_hbm, zr_hbm, id_hbm,
          acc_out, deg_out, cc_out, cnt_out,
          srci_v, dsti_v, oidx_v, didx_v, rows_v, rows2_v, identv,
          cc_v, cntv, ramp_v, ramp64_v,
          acc_sh, sem, sem2, sem3, sem4):
        cid = lax.axis_index("c")
        sid = lax.axis_index("s")
        wid = sid * NC + cid
        lane = lax.iota(jnp.int32, 16)
        zpt = N_PAD // NS
        pltpu.sync_copy(zr_hbm, rows_v)

        def fill_ramp(base):
            for kk in range(ECH // 16):
                ramp_v[pl.ds(kk * 16, 16)] = base + kk * 16 + lane

        for q in A_BASES:
            fill_ramp(sid * zpt + q)
            pltpu.sync_copy(rows_v, acc_sh.at[ramp_v])

        @pl.when(sid == 0)
        def _init_ext():
            for q in EXT_BASES:
                fill_ramp(N_PAD + q)
                pltpu.sync_copy(rows_v, acc_sh.at[ramp_v])
            pltpu.sync_copy(id_hbm, identv)
            for half in range(2):
                for kk in range(4):
                    ramp64_v[pl.ds(kk * 16, 16)] = (IDENT_BASE + half * 64 +
                                                    kk * 16 + lane)
                pltpu.sync_copy(identv.at[pl.ds(half * 64, 64)],
                                acc_sh.at[ramp64_v])

        plsc.subcore_barrier()

        ebase = wid * EPW

        def chunk(i, cnt):
            base = ebase + i * ECH
            pltpu.sync_copy(src_hbm.at[pl.ds(base, ECH)], srci_v)
            pltpu.sync_copy(dst_hbm.at[pl.ds(base, ECH)], dsti_v)
            gat = pltpu.async_copy(x_hbm.at[srci_v], rows_v, sem)
            for kk in range(ECH // 16):
                d16 = dsti_v[pl.ds(kk * 16, 16)]
                oidx_v[pl.ds(kk * 16, 16)] = IDENT_BASE + (d16 & 127)
                didx_v[pl.ds(kk * 16, 16)] = (
                    DEG_BASE + lax.shift_right_logical(d16, 7))
            gat2 = pltpu.async_copy(acc_sh.at[oidx_v], rows2_v, sem2)
            gat.wait()
            sc1 = pltpu.async_copy(rows_v, acc_sh.at[dsti_v], sem3, add=True)
            gat2.wait()
            sc2 = pltpu.async_copy(rows2_v, acc_sh.at[didx_v], sem4,
                                   add=True)
            for kk in range(ECH // 16):
                s16 = srci_v[pl.ds(kk * 16, 16)]
                d16 = dsti_v[pl.ds(kk * 16, 16)]
                m = d16 < TRASH
                pref = plsc.cumsum(m.astype(jnp.int32))
                pos = jnp.where(m, cnt + pref - 1, CAP - 16 + lane)
                pk = lax.shift_left(d16, 18) | s16
                plsc.store_scatter(cc_v, [pos], pk)
                cnt = cnt + jnp.max(pref)
            sc1.wait()
            sc2.wait()
            return cnt

        cnt = lax.fori_loop(0, NCH, chunk, jnp.int32(0))
        for kk in range(ECH // 16):
            plsc.store_scatter(cc_v, [cnt + kk * 16 + lane],
                               jnp.full((16,), TRASH << 18, jnp.int32))
        pltpu.sync_copy(cc_v, cc_out.at[pl.ds(wid * CAP, CAP)])
        cntv[...] = jnp.full((16,), cnt, jnp.int32)
        pltpu.sync_copy(cntv, cnt_out.at[pl.ds(wid * 16, 16)])
        plsc.subcore_barrier()
        opt = N_PAD // NS
        for q in A_BASES:
            fill_ramp(sid * opt + q)
            pltpu.async_copy(acc_sh.at[ramp_v], rows_v, sem).wait()
            pltpu.sync_copy(rows_v,
                            acc_out.at[cid, pl.ds(sid * opt + q, ECH)])

        @pl.when(sid == 0)
        def _read_deg():
            fill_ramp(DEG_BASE)
            pltpu.async_copy(acc_sh.at[ramp_v], rows_v, sem).wait()
            pltpu.sync_copy(rows_v, deg_out.at[cid])

    return k(x, src, dst, zrows, ident)


def _sc_phase_c(h1a, h1b, cc, cnts, z2):
    mesh = plsc.VectorSubcoreMesh(core_axis_name="c", subcore_axis_name="s")
    out_type = (
        jax.ShapeDtypeStruct((NC, BATCH, HC // 2), jnp.float32),
        jax.ShapeDtypeStruct((NC, BATCH, HC // 2), jnp.float32),
    )
    scratch = [
        pltpu.VMEM((ECH,), jnp.int32),
        pltpu.VMEM((ECH,), jnp.int32),
        pltpu.VMEM((ECH,), jnp.int32),
        pltpu.VMEM((ECH, HC // 2), jnp.float32),
        pltpu.VMEM((ECH, HC // 2), jnp.float32),
        pltpu.VMEM((16,), jnp.int32),
        pltpu.VMEM_SHARED((N2_PAD, HC // 2), jnp.float32),
        pltpu.VMEM_SHARED((N2_PAD, HC // 2), jnp.float32),
        pltpu.SemaphoreType.DMA,
        pltpu.SemaphoreType.DMA,
        pltpu.SemaphoreType.DMA,
        pltpu.SemaphoreType.DMA,
    ]

    @functools.partial(pl.kernel, out_type=out_type, mesh=mesh,
                       scratch_types=scratch,
                       compiler_params=pltpu.CompilerParams(
                           needs_layout_passes=False))
    def k(h1a_hbm, h1b_hbm, cc_hbm, cnt_hbm, z2_hbm,
          acc2a_out, acc2b_out,
          ccj_v, srcj_v, dstj_v, rowsa_v, rowsb_v, cntv, acc2a_sh, acc2b_sh,
          sema, semb, semc, semd):
        cid = lax.axis_index("c")
        sid = lax.axis_index("s")
        wid = sid * NC + cid
        zpt = N2_PAD // NS
        pltpu.sync_copy(z2_hbm, acc2a_sh.at[pl.ds(sid * zpt, zpt)])
        pltpu.sync_copy(z2_hbm, acc2b_sh.at[pl.ds(sid * zpt, zpt)])
        pltpu.sync_copy(cnt_hbm.at[pl.ds(wid * 16, 16)], cntv)
        plsc.subcore_barrier()
        cnt = jnp.max(cntv[...])
        nch = (cnt + (ECH - 1)) // ECH

        def chunk(j, carry):
            pltpu.sync_copy(cc_hbm.at[pl.ds(wid * CAP + j * ECH, ECH)],
                            ccj_v)
            for kk in range(ECH // 16):
                pk = ccj_v[pl.ds(kk * 16, 16)]
                srcj_v[pl.ds(kk * 16, 16)] = pk & ((1 << 18) - 1)
                dstj_v[pl.ds(kk * 16, 16)] = lax.shift_right_logical(pk, 18)
            ca = pltpu.async_copy(h1a_hbm.at[srcj_v], rowsa_v, sema)
            cb = pltpu.async_copy(h1b_hbm.at[srcj_v], rowsb_v, semb)
            ca.wait()
            sa = pltpu.async_copy(rowsa_v, acc2a_sh.at[dstj_v], semc,
                                  add=True)
            cb.wait()
            sb = pltpu.async_copy(rowsb_v, acc2b_sh.at[dstj_v], semd,
                                  add=True)
            sa.wait()
            sb.wait()
            return carry

        lax.fori_loop(0, nch, chunk, jnp.int32(0))
        plsc.subcore_barrier()
        opt = BATCH // NS
        pltpu.sync_copy(acc2a_sh.at[pl.ds(sid * opt, opt)],
                        acc2a_out.at[cid, pl.ds(sid * opt, opt)])
        pltpu.sync_copy(acc2b_sh.at[pl.ds(sid * opt, opt)],
                        acc2b_out.at[cid, pl.ds(sid * opt, opt)])

    return k(h1a, h1b, cc, cnts, z2)


def _tc_h1(x, acc, deg, Ws1, Wn1, bs1):
    BLK = 1000
    H2 = HC // 2

    def body(x_ref, a0_ref, a1_ref, d_ref, ws_ref, wn_ref, b_ref,
             oa_ref, ob_ref):
        d = jnp.maximum(d_ref[...][:, 0:1], 1.0)
        agg = (a0_ref[...] + a1_ref[...]) / d
        h = jnp.dot(x_ref[...], ws_ref[...],
                    preferred_element_type=jnp.float32)
        h = h + jnp.dot(agg, wn_ref[...], preferred_element_type=jnp.float32)
        h = jnp.maximum(h + b_ref[...], 0.0)
        oa_ref[...] = h[:, :H2]
        ob_ref[...] = h[:, H2:]

    return pl.pallas_call(
        body,
        grid=(N_NODES // BLK,),
        in_specs=[
            pl.BlockSpec((BLK, D_IN), lambda i: (i, 0)),
            pl.BlockSpec((BLK, D_IN), lambda i: (i, 0)),
            pl.BlockSpec((BLK, D_IN), lambda i: (i, 0)),
            pl.BlockSpec((BLK, DEGW), lambda i: (i, 0)),
            pl.BlockSpec((D_IN, HC), lambda i: (0, 0)),
            pl.BlockSpec((D_IN, HC), lambda i: (0, 0)),
            pl.BlockSpec((1, HC), lambda i: (0, 0)),
        ],
        out_specs=[
            pl.BlockSpec((BLK, H2), lambda i: (i, 0)),
            pl.BlockSpec((BLK, H2), lambda i: (i, 0)),
        ],
        out_shape=[
            jax.ShapeDtypeStruct((N_NODES, H2), jnp.float32),
            jax.ShapeDtypeStruct((N_NODES, H2), jnp.float32),
        ],
    )(x, acc[0], acc[1], deg, Ws1, Wn1, bs1.reshape(1, -1))


def _tc_final(E, K, M0, M1, Sc, W1p, b1p, W2p, b2p, W3p, b3p, W4f, b4f,
              h1a, h1b, a2a0, a2a1, a2b0, a2b1, degb,
              Ws2, Wn2, bs2, Wm1, bm1, Wm2p, bm2p, Wm3f):
    BLK = 128
    L = E.shape[1]
    LD = K.shape[1]
    H2 = HC // 2

    def body(e_ref, k_ref, m0_ref, m1_ref, sc_ref,
             w1_ref, b1_ref, w2_ref, b2_ref, w3_ref, b3_ref, w4_ref, b4_ref,
             h1a_ref, h1b_ref, a2a0_ref, a2a1_ref, a2b0_ref, a2b1_ref,
             d_ref,
             ws2_ref, wn2_ref, bs2_ref, wm1_ref, bm1_ref, wm2_ref, bm2_ref,
             wm3_ref, o_ref):
        dot = lambda a, b: jnp.dot(a, b, preferred_element_type=jnp.float32)
        e = e_ref[...]
        q = dot(e, k_ref[...]) * dot(e, m0_ref[...]) * dot(e, m1_ref[...])
        pooled = dot(q, sc_ref[...])
        h = jnp.maximum(dot(pooled, w1_ref[...]) + b1_ref[...], 0.0)
        h = jnp.maximum(dot(h, w2_ref[...]) + b2_ref[...], 0.0)
        h = jnp.maximum(dot(h, w3_ref[...]) + b3_ref[...], 0.0)
        hy = dot(h, w4_ref[...]) + b4_ref[...]
        d = jnp.maximum(d_ref[...][:, 0:1], 1.0)
        agg2a = (a2a0_ref[...] + a2a1_ref[...]) / d
        agg2b = (a2b0_ref[...] + a2b1_ref[...]) / d
        ws2 = ws2_ref[...]
        wn2 = wn2_ref[...]
        g = (dot(h1a_ref[...], ws2[:H2]) + dot(h1b_ref[...], ws2[H2:]) +
             dot(agg2a, wn2[:H2]) + dot(agg2b, wn2[H2:]) + bs2_ref[...])
        g = jnp.maximum(g, 0.0)
        m = jnp.maximum(dot(g, wm1_ref[...]) + bm1_ref[...], 0.0)
        m = jnp.maximum(dot(m, wm2_ref[...]) + bm2_ref[...], 0.0)
        o_ref[...] = hy + dot(m, wm3_ref[...])

    full = lambda r, c: pl.BlockSpec((r, c), lambda i: (0, 0))
    row = lambda c: pl.BlockSpec((BLK, c), lambda i: (i, 0))
    return pl.pallas_call(
        body,
        grid=(BATCH // BLK,),
        in_specs=[
            row(L),
            full(L, LD), full(L, LD), full(L, LD), full(LD, 128),
            full(128, 1024), full(1, 1024),
            full(1024, 512), full(1, 512),
            full(512, 256), full(1, 256),
            full(256, 128), full(1, 128),
            row(H2), row(H2),
            row(H2), row(H2), row(H2), row(H2),
            row(DEGW),
            full(HC, HC), full(HC, HC), full(1, HC),
            full(HC, 128), full(1, 128),
            full(128, 128), full(1, 128),
            full(128, 128),
        ],
        out_specs=pl.BlockSpec((BLK, 128), lambda i: (i, 0)),
        out_shape=jax.ShapeDtypeStruct((BATCH, 128), jnp.float32),
    )(E, K, M0, M1, Sc, W1p, b1p, W2p, b2p, W3p, b3p, W4f, b4f,
      h1a, h1b, a2a0, a2a1, a2b0, a2b1, degb,
      Ws2, Wn2, bs2, Wm1, bm1, Wm2p, bm2p, Wm3f)


def kernel(x, edge_index, inputs_embeds, batch_size, Wm_in, filt, Wg, Wv,
           W1, b1, W2, b2, W3, b3, W4, b4,
           Ws1, Wn1, bs1, Ws2, Wn2, bs2,
           Wm1, bm1, Wm2, bm2, Wm3, bm3, Wfc, bfc):
    f32 = jnp.float32
    src = edge_index[0]
    dst = edge_index[1]

    zrows = jnp.zeros((ECH, D_IN), f32)
    ident = jnp.eye(128, dtype=f32)
    acc, degblk, cc, cnts = _sc_phase_a(x, src, dst, zrows, ident)
    deg_flat = (degblk[0] + degblk[1]).reshape(-1)[:N_NODES]
    degp = jnp.pad(deg_flat[:, None], ((0, 0), (0, DEGW - 1)))

    h1a, h1b = _tc_h1(x, acc[:, :N_NODES], degp, Ws1, Wn1, bs1)

    z2 = jnp.zeros((N2_PAD // NS, HC // 2), f32)
    acc2a, acc2b = _sc_phase_c(h1a, h1b, cc, cnts, z2)

    L = inputs_embeds.shape[1]
    DM = Wv.shape[0]
    r = jnp.arange(L)
    dmat = r[None, :] - r[:, None]
    f0 = filt[0]
    f1 = filt[1]
    M0 = jnp.where((dmat >= 0)[:, :, None], f0[dmat % L],
                   0.0).reshape(L, L * DM)
    M1 = jnp.where((dmat.T >= 0)[:, :, None], f1[dmat.T % L],
                   0.0).reshape(L, L * DM)
    K = jnp.repeat(jnp.eye(L, dtype=f32), DM, axis=1)
    v = Wm_in @ Wv
    g0 = Wm_in @ Wg[0]
    g1 = Wm_in @ Wg[1]
    coef = g0 * g1 * v / L
    Sc = jnp.pad(jnp.tile(jnp.diag(coef), (L, 1)), ((0, 0), (0, 128 - DM)))
    W1p = jnp.pad(W1, ((0, 128 - DM), (0, 24)))
    b1p = jnp.pad(b1, (0, 24)).reshape(1, -1)
    W2p = jnp.pad(W2, ((0, 24), (0, 12)))
    b2p = jnp.pad(b2, (0, 12)).reshape(1, -1)
    W3p = jnp.pad(W3, ((0, 12), (0, 56)))
    b3p = jnp.pad(b3, (0, 56)).reshape(1, -1)
    wfc0 = Wfc[0, 0]
    wfc1 = Wfc[1, 0]
    W4f = jnp.pad(W4 * wfc0, ((0, 56), (0, 127)))
    b4f = jnp.zeros((1, 128), f32).at[0, 0].set(
        wfc0 * b4[0] + wfc1 * bm3[0] + bfc[0])
    Wm2p = jnp.pad(Wm2, ((0, 0), (0, 64)))
    bm2p = jnp.pad(bm2, (0, 64)).reshape(1, -1)
    Wm3f = jnp.pad(Wm3 * wfc1, ((0, 64), (0, 127)))

    out2d = _tc_final(inputs_embeds, K, M0, M1, Sc, W1p, b1p, W2p, b2p,
                      W3p, b3p, W4f, b4f,
                      h1a[:BATCH], h1b[:BATCH],
                      acc2a[0], acc2a[1], acc2b[0], acc2b[1],
                      degp[:BATCH],
                      Ws2, Wn2, bs2.reshape(1, -1), Wm1, bm1.reshape(1, -1),
                      Wm2p, bm2p, Wm3f)
    return out2d[:, 0]
